# baseline (speedup 1.0000x reference)
import functools

import jax
import jax.numpy as jnp
from jax import lax
from jax.experimental import pallas as pl
from jax.experimental.pallas import tpu as pltpu
from jax.experimental.pallas import tpu_sc as plsc

NC = 2
NS = 16
NW = NC * NS
EB = 80
CW = 32
CNTW = 8
NBLK = 512


def _sc_mesh():
    return plsc.VectorSubcoreMesh(core_axis_name="c", subcore_axis_name="s")


_SC_PARAMS = pltpu.CompilerParams(use_tc_tiling_on_sc=False)


def _gather_rows(table, idx, n_pad, d):
    rows_per_w = n_pad // NW
    nb = rows_per_w // EB

    @functools.partial(
        pl.kernel,
        mesh=_sc_mesh(),
        out_type=jax.ShapeDtypeStruct((n_pad, d), jnp.float32),
        compiler_params=_SC_PARAMS,
        scratch_types=[
            pltpu.VMEM((EB,), jnp.int32),
            pltpu.VMEM((EB, d), jnp.float32),
            pltpu.SemaphoreType.DMA,
        ],
    )
    def k(table_hbm, idx_hbm, out_hbm, idx_v, rows_v, sem):
        wid = lax.axis_index("c") * NS + lax.axis_index("s")
        base = wid * rows_per_w

        def body(b, carry):
            off = base + b * EB
            pltpu.sync_copy(idx_hbm.at[pl.ds(off, EB)], idx_v)
            pltpu.async_copy(table_hbm.at[idx_v], rows_v, sem).wait()
            pltpu.sync_copy(rows_v, out_hbm.at[pl.ds(off, EB)])
            return carry

        lax.fori_loop(0, nb, body, 0)

    return k(table, idx)


def _sc_aggregate(tabs, src, comb, n_pad, et):
    e = src.shape[0]
    epw = e // NW
    nb = epw // EB
    rows = et * n_pad
    rpt = rows // NS
    nch = len(tabs)

    zb = 320
    assert rpt % zb == 0
    zeros_acc = jnp.zeros((zb, CW), jnp.float32)
    zeros_cnt = jnp.zeros((zb, CNTW), jnp.float32)
    ones = jnp.ones((EB, CNTW), jnp.float32)

    nbuf = 5
    assert nb % nbuf == 0

    @functools.partial(
        pl.kernel,
        mesh=_sc_mesh(),
        out_type=(
            jax.ShapeDtypeStruct((NC, et, n_pad, nch, CW), jnp.float32),
            jax.ShapeDtypeStruct((NC, rows, CNTW), jnp.float32),
        ),
        compiler_params=_SC_PARAMS,
        scratch_types=(
            [pltpu.VMEM_SHARED((rows, CW), jnp.float32),
             pltpu.VMEM_SHARED((rows, CNTW), jnp.float32)]
            + [pltpu.VMEM((EB,), jnp.int32)] * (2 * nbuf)
            + [pltpu.VMEM((EB, CW), jnp.float32)] * nbuf
            + [pltpu.VMEM((EB, CNTW), jnp.float32)]
            + [pltpu.VMEM((zb, CW), jnp.float32)]
            + [pltpu.VMEM((zb, CNTW), jnp.float32)]
            + [pltpu.SemaphoreType.DMA] * (3 * nbuf)
        ),
    )
    def k(t0, t1, t2, t3, src_hbm, comb_hbm, za_hbm, zc_hbm, ones_hbm,
          p_out, cnt_out, *scr):
        acc_sh, cnt_sh = scr[0], scr[1]
        srcs = scr[2:2 + nbuf]
        combs = scr[2 + nbuf:2 + 2 * nbuf]
        rows_v = scr[2 + 2 * nbuf:2 + 3 * nbuf]
        ones_v = scr[2 + 3 * nbuf]
        zeros_av = scr[3 + 3 * nbuf]
        zeros_cv = scr[4 + 3 * nbuf]
        isem = scr[5 + 3 * nbuf:5 + 4 * nbuf]
        gsem = scr[5 + 4 * nbuf:5 + 5 * nbuf]
        ssem = scr[5 + 5 * nbuf:5 + 6 * nbuf]
        ci = lax.axis_index("c")
        si = lax.axis_index("s")
        wid = ci * NS + si
        ebase = wid * epw
        pltpu.sync_copy(ones_hbm, ones_v)
        pltpu.sync_copy(za_hbm, zeros_av)
        pltpu.sync_copy(zc_hbm, zeros_cv)
        for c, tab in enumerate((t0, t1, t2, t3)):
            for z in range(rpt // zb):
                pltpu.sync_copy(zeros_av,
                                acc_sh.at[pl.ds(si * rpt + z * zb, zb)])
                if c == 0:
                    pltpu.sync_copy(zeros_cv,
                                    cnt_sh.at[pl.ds(si * rpt + z * zb, zb)])
            plsc.subcore_barrier()

            def body(r, carry):
                base = ebase + r * (nbuf * EB)
                ihs = []
                for j in range(nbuf):
                    off = base + j * EB
                    ihs.append((
                        pltpu.async_copy(src_hbm.at[pl.ds(off, EB)],
                                         srcs[j], isem[j]),
                        pltpu.async_copy(comb_hbm.at[pl.ds(off, EB)],
                                         combs[j], isem[j])))
                ghs = []
                for j in range(nbuf):
                    ihs[j][0].wait()
                    ihs[j][1].wait()
                    ghs.append(pltpu.async_copy(tab.at[srcs[j]],
                                                rows_v[j], gsem[j]))
                shs = []
                for j in range(nbuf):
                    ghs[j].wait()
                    shs.append(pltpu.async_copy(
                        rows_v[j], acc_sh.at[combs[j]], ssem[j], add=True))
                    if c == 0:
                        shs.append(pltpu.async_copy(
                            ones_v, cnt_sh.at[combs[j]], ssem[j], add=True))
                for h in shs:
                    h.wait()
                return carry

            lax.fori_loop(0, nb // nbuf, body, 0)
            plsc.subcore_barrier()
            for z in range(rpt // zb):
                o = si * rpt + z * zb
                t = o // n_pad
                node_lo = o % n_pad
                pltpu.sync_copy(acc_sh.at[pl.ds(o, zb)],
                                p_out.at[ci, t, pl.ds(node_lo, zb), c])
                if c == 0:
                    pltpu.sync_copy(cnt_sh.at[pl.ds(o, zb)],
                                    cnt_out.at[ci, pl.ds(o, zb)])
            plsc.subcore_barrier()

    return k(tabs[0], tabs[1], tabs[2], tabs[3], src, comb,
             zeros_acc, zeros_cnt, ones)


def _tc_layer(p8, cnt8, h, nt, rel_w, root_w, root_b, n_pad, d, last):
    et = rel_w.shape[0]
    ntyp = root_w.shape[0]
    grid = (n_pad // NBLK,)
    rb = root_b.reshape(ntyp, 1, d)

    def body(p_ref, c_ref, h_ref, nt_ref, rw_ref, tw_ref, tb_ref, *o_refs):
        pv = p_ref[...]
        cv = c_ref[...]
        acc = jnp.zeros((NBLK, d), jnp.float32)
        for t in range(et):
            ct = cv[:, t:t + 1] + cv[:, et + t:et + t + 1]
            inv = 1.0 / jnp.maximum(ct, 1.0)
            at = (pv[t] + pv[et + t]) * inv
            acc = acc + jnp.dot(at, rw_ref[t],
                                preferred_element_type=jnp.float32)
        hv = h_ref[...]
        ntv = nt_ref[...]
        for i in range(ntyp):
            ri = jnp.dot(hv, tw_ref[i],
                         preferred_element_type=jnp.float32) + tb_ref[i]
            acc = acc + jnp.where(ntv == i, ri, 0.0)
        if last:
            m = jnp.max(acc, axis=-1, keepdims=True)
            x = acc - m
            o_refs[0][...] = x - jnp.log(
                jnp.sum(jnp.exp(x), axis=-1, keepdims=True))
        else:
            res = jnp.maximum(acc, 0.0)
            o_refs[0][...] = res
            for c in range(d // CW):
                o_refs[1][c] = res[:, c * CW:(c + 1) * CW]

    return pl.pallas_call(
        body,
        grid=grid,
        in_specs=[
            pl.BlockSpec((2 * et, NBLK, d), lambda i: (0, i, 0)),
            pl.BlockSpec((NBLK, 2 * et), lambda i: (i, 0)),
            pl.BlockSpec((NBLK, d), lambda i: (i, 0)),
            pl.BlockSpec((NBLK, 1), lambda i: (i, 0)),
            pl.BlockSpec((et, d, d), lambda i: (0, 0, 0)),
            pl.BlockSpec((ntyp, d, d), lambda i: (0, 0, 0)),
            pl.BlockSpec((ntyp, 1, d), lambda i: (0, 0, 0)),
        ],
        out_specs=(
            [pl.BlockSpec((NBLK, d), lambda i: (i, 0))] if last else
            [pl.BlockSpec((NBLK, d), lambda i: (i, 0)),
             pl.BlockSpec((d // CW, NBLK, CW), lambda i: (0, i, 0))]),
        out_shape=(
            [jax.ShapeDtypeStruct((n_pad, d), jnp.float32)] if last else
            [jax.ShapeDtypeStruct((n_pad, d), jnp.float32),
             jax.ShapeDtypeStruct((d // CW, n_pad, CW), jnp.float32)]),
    )(p8, cnt8, h, nt, rel_w, root_w, rb)


def kernel(x_dict, edge_index, edge_type, node_type, local_node_idx, emb1,
           rel_W1, root_W1, root_b1, rel_W2, root_W2, root_b2):
    n = node_type.shape[0]
    npt = x_dict.shape[0]
    d = x_dict.shape[1]
    et = rel_W1.shape[0]
    unit = max(NBLK, NW * EB)
    n_pad = ((n + unit - 1) // unit) * unit

    src = edge_index[0].astype(jnp.int32)
    dst = edge_index[1].astype(jnp.int32)
    comb = (edge_type.astype(jnp.int32) * n_pad + dst)
    table = jnp.concatenate([x_dict, emb1], axis=0)
    idx0 = local_node_idx.astype(jnp.int32) + node_type.astype(jnp.int32) * npt
    idx0 = jnp.pad(idx0, (0, n_pad - n))
    ntp = jnp.pad(node_type.astype(jnp.int32), (0, n_pad - n)).reshape(n_pad, 1)

    h = _gather_rows(table, idx0, n_pad, d)
    tabs = [h[:, c * CW:(c + 1) * CW] for c in range(d // CW)]

    for rel_w, root_w, root_b, last in (
            (rel_W1, root_W1, root_b1, False),
            (rel_W2, root_W2, root_b2, True)):
        p, cnt = _sc_aggregate(tabs, src, comb, n_pad, et)
        p8 = p.reshape(NC * et, n_pad, d)
        cnt8 = cnt[:, :, 0].reshape(NC, et, n_pad).transpose(2, 0, 1).reshape(
            n_pad, NC * et)
        outs = _tc_layer(p8, cnt8, h, ntp, rel_w, root_w, root_b, n_pad, d,
                         last)
        if last:
            h = outs[0]
        else:
            h, hc = outs
            tabs = [hc[c] for c in range(d // CW)]

    return h[:n]


# speedup vs baseline: 1.0724x; 1.0724x over previous
import functools

import jax
import jax.numpy as jnp
from jax import lax
from jax.experimental import pallas as pl
from jax.experimental.pallas import tpu as pltpu
from jax.experimental.pallas import tpu_sc as plsc

NC = 2
NS = 16
NW = NC * NS
EB = 80
CW = 32
CNTW = 8
NBLK = 512


def _sc_mesh():
    return plsc.VectorSubcoreMesh(core_axis_name="c", subcore_axis_name="s")


_SC_PARAMS = pltpu.CompilerParams(use_tc_tiling_on_sc=False)


def _gather_rows(table, idx, n_pad, d):
    rows_per_w = n_pad // NW
    nb = rows_per_w // EB

    @functools.partial(
        pl.kernel,
        mesh=_sc_mesh(),
        out_type=jax.ShapeDtypeStruct((n_pad, d), jnp.float32),
        compiler_params=_SC_PARAMS,
        scratch_types=[
            pltpu.VMEM((EB,), jnp.int32),
            pltpu.VMEM((EB, d), jnp.float32),
            pltpu.SemaphoreType.DMA,
        ],
    )
    def k(table_hbm, idx_hbm, out_hbm, idx_v, rows_v, sem):
        wid = lax.axis_index("c") * NS + lax.axis_index("s")
        base = wid * rows_per_w

        def body(b, carry):
            off = base + b * EB
            pltpu.sync_copy(idx_hbm.at[pl.ds(off, EB)], idx_v)
            pltpu.async_copy(table_hbm.at[idx_v], rows_v, sem).wait()
            pltpu.sync_copy(rows_v, out_hbm.at[pl.ds(off, EB)])
            return carry

        lax.fori_loop(0, nb, body, 0)

    return k(table, idx)


def _sc_aggregate(tabs, src, comb, n_pad, et):
    e = src.shape[0]
    epw = e // NW
    nb = epw // EB
    rows = et * n_pad
    rpt = rows // NS
    nch = len(tabs)

    zb = 320
    assert rpt % zb == 0
    zeros_acc = jnp.zeros((zb, CW), jnp.float32)
    zeros_cnt = jnp.zeros((zb, CNTW), jnp.float32)
    ones = jnp.ones((EB, CNTW), jnp.float32)

    nbuf = 5
    assert nb % nbuf == 0

    @functools.partial(
        pl.kernel,
        mesh=_sc_mesh(),
        out_type=(
            jax.ShapeDtypeStruct((NC, nch, rows, CW), jnp.float32),
            jax.ShapeDtypeStruct((NC, rows, CNTW), jnp.float32),
        ),
        compiler_params=_SC_PARAMS,
        scratch_types=(
            [pltpu.VMEM_SHARED((rows, CW), jnp.float32),
             pltpu.VMEM_SHARED((rows, CNTW), jnp.float32)]
            + [pltpu.VMEM((EB,), jnp.int32)] * (2 * nbuf)
            + [pltpu.VMEM((EB, CW), jnp.float32)] * nbuf
            + [pltpu.VMEM((EB, CNTW), jnp.float32)]
            + [pltpu.VMEM((zb, CW), jnp.float32)]
            + [pltpu.VMEM((zb, CNTW), jnp.float32)]
            + [pltpu.SemaphoreType.DMA] * (3 * nbuf)
        ),
    )
    def k(t0, t1, t2, t3, src_hbm, comb_hbm, za_hbm, zc_hbm, ones_hbm,
          p_out, cnt_out, *scr):
        acc_sh, cnt_sh = scr[0], scr[1]
        srcs = scr[2:2 + nbuf]
        combs = scr[2 + nbuf:2 + 2 * nbuf]
        rows_v = scr[2 + 2 * nbuf:2 + 3 * nbuf]
        ones_v = scr[2 + 3 * nbuf]
        zeros_av = scr[3 + 3 * nbuf]
        zeros_cv = scr[4 + 3 * nbuf]
        isem = scr[5 + 3 * nbuf:5 + 4 * nbuf]
        gsem = scr[5 + 4 * nbuf:5 + 5 * nbuf]
        ssem = scr[5 + 5 * nbuf:5 + 6 * nbuf]
        ci = lax.axis_index("c")
        si = lax.axis_index("s")
        wid = ci * NS + si
        ebase = wid * epw
        pltpu.sync_copy(ones_hbm, ones_v)
        pltpu.sync_copy(za_hbm, zeros_av)
        pltpu.sync_copy(zc_hbm, zeros_cv)
        for c, tab in enumerate((t0, t1, t2, t3)):
            for z in range(rpt // zb):
                pltpu.sync_copy(zeros_av,
                                acc_sh.at[pl.ds(si * rpt + z * zb, zb)])
                if c == 0:
                    pltpu.sync_copy(zeros_cv,
                                    cnt_sh.at[pl.ds(si * rpt + z * zb, zb)])
            plsc.subcore_barrier()

            def body(r, carry):
                base = ebase + r * (nbuf * EB)
                ihs = []
                for j in range(nbuf):
                    off = base + j * EB
                    ihs.append((
                        pltpu.async_copy(src_hbm.at[pl.ds(off, EB)],
                                         srcs[j], isem[j]),
                        pltpu.async_copy(comb_hbm.at[pl.ds(off, EB)],
                                         combs[j], isem[j])))
                ghs = []
                for j in range(nbuf):
                    ihs[j][0].wait()
                    ihs[j][1].wait()
                    ghs.append(pltpu.async_copy(tab.at[srcs[j]],
                                                rows_v[j], gsem[j]))
                shs = []
                for j in range(nbuf):
                    ghs[j].wait()
                    shs.append(pltpu.async_copy(
                        rows_v[j], acc_sh.at[combs[j]], ssem[j], add=True))
                    if c == 0:
                        shs.append(pltpu.async_copy(
                            ones_v, cnt_sh.at[combs[j]], ssem[j], add=True))
                for h in shs:
                    h.wait()
                return carry

            lax.fori_loop(0, nb // nbuf, body, 0)
            plsc.subcore_barrier()
            for z in range(rpt // zb):
                o = si * rpt + z * zb
                pltpu.sync_copy(acc_sh.at[pl.ds(o, zb)],
                                p_out.at[ci, c, pl.ds(o, zb)])
                if c == 0:
                    pltpu.sync_copy(cnt_sh.at[pl.ds(o, zb)],
                                    cnt_out.at[ci, pl.ds(o, zb)])
            plsc.subcore_barrier()

    return k(tabs[0], tabs[1], tabs[2], tabs[3], src, comb,
             zeros_acc, zeros_cnt, ones)


def _tc_layer(p8, cnt8, h, nt, rel_w, root_w, root_b, n_pad, d, last):
    et = rel_w.shape[0]
    ntyp = root_w.shape[0]
    grid = (n_pad // NBLK,)
    rb = root_b.reshape(ntyp, 1, d)

    def body(p_ref, c_ref, h_ref, nt_ref, rw_ref, tw_ref, tb_ref, *o_refs):
        pv = p_ref[...]
        cv = c_ref[...]
        acc = jnp.zeros((NBLK, d), jnp.float32)
        for t in range(et):
            ct = cv[:, t:t + 1] + cv[:, et + t:et + t + 1]
            inv = 1.0 / jnp.maximum(ct, 1.0)
            at = (pv[t] + pv[et + t]) * inv
            acc = acc + jnp.dot(at, rw_ref[t],
                                preferred_element_type=jnp.float32)
        hv = h_ref[...]
        ntv = nt_ref[...]
        for i in range(ntyp):
            ri = jnp.dot(hv, tw_ref[i],
                         preferred_element_type=jnp.float32) + tb_ref[i]
            acc = acc + jnp.where(ntv == i, ri, 0.0)
        if last:
            m = jnp.max(acc, axis=-1, keepdims=True)
            x = acc - m
            o_refs[0][...] = x - jnp.log(
                jnp.sum(jnp.exp(x), axis=-1, keepdims=True))
        else:
            res = jnp.maximum(acc, 0.0)
            o_refs[0][...] = res
            for c in range(d // CW):
                o_refs[1][c] = res[:, c * CW:(c + 1) * CW]

    return pl.pallas_call(
        body,
        grid=grid,
        in_specs=[
            pl.BlockSpec((2 * et, NBLK, d), lambda i: (0, i, 0)),
            pl.BlockSpec((NBLK, 2 * et), lambda i: (i, 0)),
            pl.BlockSpec((NBLK, d), lambda i: (i, 0)),
            pl.BlockSpec((NBLK, 1), lambda i: (i, 0)),
            pl.BlockSpec((et, d, d), lambda i: (0, 0, 0)),
            pl.BlockSpec((ntyp, d, d), lambda i: (0, 0, 0)),
            pl.BlockSpec((ntyp, 1, d), lambda i: (0, 0, 0)),
        ],
        out_specs=(
            [pl.BlockSpec((NBLK, d), lambda i: (i, 0))] if last else
            [pl.BlockSpec((NBLK, d), lambda i: (i, 0)),
             pl.BlockSpec((d // CW, NBLK, CW), lambda i: (0, i, 0))]),
        out_shape=(
            [jax.ShapeDtypeStruct((n_pad, d), jnp.float32)] if last else
            [jax.ShapeDtypeStruct((n_pad, d), jnp.float32),
             jax.ShapeDtypeStruct((d // CW, n_pad, CW), jnp.float32)]),
    )(p8, cnt8, h, nt, rel_w, root_w, rb)


def kernel(x_dict, edge_index, edge_type, node_type, local_node_idx, emb1,
           rel_W1, root_W1, root_b1, rel_W2, root_W2, root_b2):
    n = node_type.shape[0]
    npt = x_dict.shape[0]
    d = x_dict.shape[1]
    et = rel_W1.shape[0]
    unit = max(NBLK, NW * EB)
    n_pad = ((n + unit - 1) // unit) * unit

    src = edge_index[0].astype(jnp.int32)
    dst = edge_index[1].astype(jnp.int32)
    comb = (edge_type.astype(jnp.int32) * n_pad + dst)
    table = jnp.concatenate([x_dict, emb1], axis=0)
    idx0 = local_node_idx.astype(jnp.int32) + node_type.astype(jnp.int32) * npt
    idx0 = jnp.pad(idx0, (0, n_pad - n))
    ntp = jnp.pad(node_type.astype(jnp.int32), (0, n_pad - n)).reshape(n_pad, 1)

    h = _gather_rows(table, idx0, n_pad, d)
    tabs = [h[:, c * CW:(c + 1) * CW] for c in range(d // CW)]

    for rel_w, root_w, root_b, last in (
            (rel_W1, root_W1, root_b1, False),
            (rel_W2, root_W2, root_b2, True)):
        p, cnt = _sc_aggregate(tabs, src, comb, n_pad, et)
        p8 = (p.reshape(NC, d // CW, et, n_pad, CW)
               .transpose(0, 2, 3, 1, 4)
               .reshape(NC * et, n_pad, d))
        cnt8 = cnt[:, :, 0].reshape(NC, et, n_pad).transpose(2, 0, 1).reshape(
            n_pad, NC * et)
        outs = _tc_layer(p8, cnt8, h, ntp, rel_w, root_w, root_b, n_pad, d,
                         last)
        if last:
            h = outs[0]
        else:
            h, hc = outs
            tabs = [hc[c] for c in range(d // CW)]

    return h[:n]


# speedup vs baseline: 1.1540x; 1.0761x over previous
import functools

import jax
import jax.numpy as jnp
from jax import lax
from jax.experimental import pallas as pl
from jax.experimental.pallas import tpu as pltpu
from jax.experimental.pallas import tpu_sc as plsc

NC = 2
NS = 16
NW = NC * NS
EB = 80
CW = 32
CNTW = 8
NBLK = 512


def _sc_mesh():
    return plsc.VectorSubcoreMesh(core_axis_name="c", subcore_axis_name="s")


_SC_PARAMS = pltpu.CompilerParams(use_tc_tiling_on_sc=False)


def _gather_rows(table, idx, n_pad, d):
    rows_per_w = n_pad // NW
    nb = rows_per_w // EB

    @functools.partial(
        pl.kernel,
        mesh=_sc_mesh(),
        out_type=jax.ShapeDtypeStruct((n_pad, d), jnp.float32),
        compiler_params=_SC_PARAMS,
        scratch_types=[
            pltpu.VMEM((EB,), jnp.int32),
            pltpu.VMEM((EB, d), jnp.float32),
            pltpu.SemaphoreType.DMA,
        ],
    )
    def k(table_hbm, idx_hbm, out_hbm, idx_v, rows_v, sem):
        wid = lax.axis_index("c") * NS + lax.axis_index("s")
        base = wid * rows_per_w

        def body(b, carry):
            off = base + b * EB
            pltpu.sync_copy(idx_hbm.at[pl.ds(off, EB)], idx_v)
            pltpu.async_copy(table_hbm.at[idx_v], rows_v, sem).wait()
            pltpu.sync_copy(rows_v, out_hbm.at[pl.ds(off, EB)])
            return carry

        lax.fori_loop(0, nb, body, 0)

    return k(table, idx)


def _sc_aggregate(tabs, src, comb, n_pad, et):
    e = src.shape[0]
    epw = e // NW
    nb = epw // EB
    rows = et * n_pad
    rpt = rows // NS
    nch = len(tabs)

    zb = 320
    assert rpt % zb == 0
    zeros_acc = jnp.zeros((zb, CW), jnp.float32)
    zeros_cnt = jnp.zeros((zb, CNTW), jnp.float32)
    ones = jnp.ones((EB, CNTW), jnp.float32)

    nbuf = 5
    assert nb % nbuf == 0

    @functools.partial(
        pl.kernel,
        mesh=_sc_mesh(),
        out_type=(
            jax.ShapeDtypeStruct((NC, nch, rows, CW), jnp.float32),
            jax.ShapeDtypeStruct((NC, rows, CNTW), jnp.float32),
        ),
        compiler_params=_SC_PARAMS,
        scratch_types=(
            [pltpu.VMEM_SHARED((rows, CW), jnp.float32),
             pltpu.VMEM_SHARED((rows, CNTW), jnp.float32)]
            + [pltpu.VMEM((EB,), jnp.int32)] * (2 * nbuf)
            + [pltpu.VMEM((EB, CW), jnp.float32)] * nbuf
            + [pltpu.VMEM((EB, CNTW), jnp.float32)]
            + [pltpu.VMEM((zb, CW), jnp.float32)]
            + [pltpu.VMEM((zb, CNTW), jnp.float32)]
            + [pltpu.SemaphoreType.DMA] * (3 * nbuf)
        ),
    )
    def k(t0, t1, t2, t3, src_hbm, comb_hbm, za_hbm, zc_hbm, ones_hbm,
          p_out, cnt_out, *scr):
        acc_sh, cnt_sh = scr[0], scr[1]
        srcs = scr[2:2 + nbuf]
        combs = scr[2 + nbuf:2 + 2 * nbuf]
        rows_v = scr[2 + 2 * nbuf:2 + 3 * nbuf]
        ones_v = scr[2 + 3 * nbuf]
        zeros_av = scr[3 + 3 * nbuf]
        zeros_cv = scr[4 + 3 * nbuf]
        isem = scr[5 + 3 * nbuf:5 + 4 * nbuf]
        gsem = scr[5 + 4 * nbuf:5 + 5 * nbuf]
        ssem = scr[5 + 5 * nbuf:5 + 6 * nbuf]
        ci = lax.axis_index("c")
        si = lax.axis_index("s")
        wid = ci * NS + si
        ebase = wid * epw
        pltpu.sync_copy(ones_hbm, ones_v)
        pltpu.sync_copy(za_hbm, zeros_av)
        pltpu.sync_copy(zc_hbm, zeros_cv)
        for c, tab in enumerate((t0, t1, t2, t3)):
            for z in range(rpt // zb):
                pltpu.sync_copy(zeros_av,
                                acc_sh.at[pl.ds(si * rpt + z * zb, zb)])
                if c == 0:
                    pltpu.sync_copy(zeros_cv,
                                    cnt_sh.at[pl.ds(si * rpt + z * zb, zb)])
            plsc.subcore_barrier()

            def body(r, carry):
                base = ebase + r * (nbuf * EB)
                ihs = []
                for j in range(nbuf):
                    off = base + j * EB
                    ihs.append((
                        pltpu.async_copy(src_hbm.at[pl.ds(off, EB)],
                                         srcs[j], isem[j]),
                        pltpu.async_copy(comb_hbm.at[pl.ds(off, EB)],
                                         combs[j], isem[j])))
                ghs = []
                for j in range(nbuf):
                    ihs[j][0].wait()
                    ihs[j][1].wait()
                    ghs.append(pltpu.async_copy(tab.at[srcs[j]],
                                                rows_v[j], gsem[j]))
                shs = []
                for j in range(nbuf):
                    ghs[j].wait()
                    shs.append(pltpu.async_copy(
                        rows_v[j], acc_sh.at[combs[j]], ssem[j], add=True))
                    if c == 0:
                        shs.append(pltpu.async_copy(
                            ones_v, cnt_sh.at[combs[j]], ssem[j], add=True))
                for h in shs:
                    h.wait()
                return carry

            lax.fori_loop(0, nb // nbuf, body, 0)
            plsc.subcore_barrier()
            for z in range(rpt // zb):
                o = si * rpt + z * zb
                pltpu.sync_copy(acc_sh.at[pl.ds(o, zb)],
                                p_out.at[ci, c, pl.ds(o, zb)])
                if c == 0:
                    pltpu.sync_copy(cnt_sh.at[pl.ds(o, zb)],
                                    cnt_out.at[ci, pl.ds(o, zb)])
            plsc.subcore_barrier()

    return k(tabs[0], tabs[1], tabs[2], tabs[3], src, comb,
             zeros_acc, zeros_cnt, ones)


def _tc_layer(p8, cnt8, h, nt, rel_w, root_w, root_b, n_pad, d, last):
    et = rel_w.shape[0]
    ntyp = root_w.shape[0]
    grid = (n_pad // NBLK,)
    rb = root_b.reshape(ntyp, 1, d)

    nch = d // CW

    def body(p_ref, c_ref, h_ref, nt_ref, rw_ref, tw_ref, tb_ref, *o_refs):
        pv = p_ref[...]
        cv = c_ref[...]
        acc = jnp.zeros((NBLK, d), jnp.float32)
        for t in range(et):
            ct = cv[t, :, 0:1] + cv[et + t, :, 0:1]
            inv = 1.0 / jnp.maximum(ct, 1.0)
            at = jnp.concatenate(
                [pv[c * et + t] + pv[(nch + c) * et + t] for c in range(nch)],
                axis=-1) * inv
            acc = acc + jnp.dot(at, rw_ref[t],
                                preferred_element_type=jnp.float32)
        hv = h_ref[...]
        ntv = nt_ref[...]
        for i in range(ntyp):
            ri = jnp.dot(hv, tw_ref[i],
                         preferred_element_type=jnp.float32) + tb_ref[i]
            acc = acc + jnp.where(ntv == i, ri, 0.0)
        if last:
            m = jnp.max(acc, axis=-1, keepdims=True)
            x = acc - m
            o_refs[0][...] = x - jnp.log(
                jnp.sum(jnp.exp(x), axis=-1, keepdims=True))
        else:
            res = jnp.maximum(acc, 0.0)
            o_refs[0][...] = res
            for c in range(d // CW):
                o_refs[1][c] = res[:, c * CW:(c + 1) * CW]

    return pl.pallas_call(
        body,
        grid=grid,
        in_specs=[
            pl.BlockSpec((NC * nch * et, NBLK, CW), lambda i: (0, i, 0)),
            pl.BlockSpec((NC * et, NBLK, CNTW), lambda i: (0, i, 0)),
            pl.BlockSpec((NBLK, d), lambda i: (i, 0)),
            pl.BlockSpec((NBLK, 1), lambda i: (i, 0)),
            pl.BlockSpec((et, d, d), lambda i: (0, 0, 0)),
            pl.BlockSpec((ntyp, d, d), lambda i: (0, 0, 0)),
            pl.BlockSpec((ntyp, 1, d), lambda i: (0, 0, 0)),
        ],
        out_specs=(
            [pl.BlockSpec((NBLK, d), lambda i: (i, 0))] if last else
            [pl.BlockSpec((NBLK, d), lambda i: (i, 0)),
             pl.BlockSpec((d // CW, NBLK, CW), lambda i: (0, i, 0))]),
        out_shape=(
            [jax.ShapeDtypeStruct((n_pad, d), jnp.float32)] if last else
            [jax.ShapeDtypeStruct((n_pad, d), jnp.float32),
             jax.ShapeDtypeStruct((d // CW, n_pad, CW), jnp.float32)]),
    )(p8, cnt8, h, nt, rel_w, root_w, rb)


def kernel(x_dict, edge_index, edge_type, node_type, local_node_idx, emb1,
           rel_W1, root_W1, root_b1, rel_W2, root_W2, root_b2):
    n = node_type.shape[0]
    npt = x_dict.shape[0]
    d = x_dict.shape[1]
    et = rel_W1.shape[0]
    unit = max(NBLK, NW * EB)
    n_pad = ((n + unit - 1) // unit) * unit

    src = edge_index[0].astype(jnp.int32)
    dst = edge_index[1].astype(jnp.int32)
    comb = (edge_type.astype(jnp.int32) * n_pad + dst)
    table = jnp.concatenate([x_dict, emb1], axis=0)
    idx0 = local_node_idx.astype(jnp.int32) + node_type.astype(jnp.int32) * npt
    idx0 = jnp.pad(idx0, (0, n_pad - n))
    ntp = jnp.pad(node_type.astype(jnp.int32), (0, n_pad - n)).reshape(n_pad, 1)

    h = _gather_rows(table, idx0, n_pad, d)
    tabs = [h[:, c * CW:(c + 1) * CW] for c in range(d // CW)]

    for rel_w, root_w, root_b, last in (
            (rel_W1, root_W1, root_b1, False),
            (rel_W2, root_W2, root_b2, True)):
        p, cnt = _sc_aggregate(tabs, src, comb, n_pad, et)
        p8 = p.reshape(NC * (d // CW) * et, n_pad, CW)
        cnt8 = cnt.reshape(NC * et, n_pad, CNTW)
        outs = _tc_layer(p8, cnt8, h, ntp, rel_w, root_w, root_b, n_pad, d,
                         last)
        if last:
            h = outs[0]
        else:
            h, hc = outs
            tabs = [hc[c] for c in range(d // CW)]

    return h[:n]
